# baseline (speedup 1.0000x reference)
import functools

import jax
import jax.numpy as jnp
from jax.experimental import pallas as pl
from jax.experimental.pallas import tpu as pltpu

D_MODEL = 768
D_SAE = 24576
N_TOK = 4096
K = 64

BR = 1024
BC = 1024


def _mm_body(x_ref, w_ref, b_ref, o_ref):
    o_ref[...] = jax.lax.dot_general(
        x_ref[...], w_ref[...], (((1,), (1,)), ((), ())),
        preferred_element_type=jnp.float32) + b_ref[...]


def _matmul(x, W, b):
    nr = N_TOK // BR
    nc = D_SAE // BC
    b2 = b.reshape(1, D_SAE)
    return pl.pallas_call(
        _mm_body,
        grid=(nr, nc),
        in_specs=[
            pl.BlockSpec((BR, D_MODEL), lambda r, c: (r, 0)),
            pl.BlockSpec((BC, D_MODEL), lambda r, c: (c, 0)),
            pl.BlockSpec((1, BC), lambda r, c: (0, c)),
        ],
        out_specs=pl.BlockSpec((BR, BC), lambda r, c: (r, c)),
        out_shape=jax.ShapeDtypeStruct((N_TOK, D_SAE), jnp.float32),
    )(x, W, b2)


def kernel(x, W, b):
    pre = _matmul(x, W, b)
    vals, idx = jax.lax.top_k(pre, K)
    rows = jnp.arange(pre.shape[0])[:, None]
    out = jnp.zeros_like(pre).at[rows, idx].set(jnp.maximum(vals, 0.0))
    return out


# speedup vs baseline: 2.4112x; 2.4112x over previous
import functools

import jax
import jax.numpy as jnp
from jax import lax
from jax.experimental import pallas as pl
from jax.experimental.pallas import tpu as pltpu
from jax.experimental.pallas import tpu_sc as plsc

D_MODEL = 768
D_SAE = 24576
N_TOK = 4096
K = 64

BR = 1024
BC = 1024

NW = 32
RPW = N_TOK // NW
NV = D_SAE // 16
NEG_INF_I32 = -2147483648


def _mm_body(x_ref, w_ref, b_ref, o_ref):
    o_ref[...] = jax.lax.dot_general(
        x_ref[...], w_ref[...], (((1,), (1,)), ((), ())),
        preferred_element_type=jnp.float32) + b_ref[...]


def _matmul(x, W, b):
    nr = N_TOK // BR
    nc = D_SAE // BC
    b2 = b.reshape(1, D_SAE)
    return pl.pallas_call(
        _mm_body,
        grid=(nr, nc),
        in_specs=[
            pl.BlockSpec((BR, D_MODEL), lambda r, c: (r, 0)),
            pl.BlockSpec((BC, D_MODEL), lambda r, c: (c, 0)),
            pl.BlockSpec((1, BC), lambda r, c: (0, c)),
        ],
        out_specs=pl.BlockSpec((BR, BC), lambda r, c: (r, c)),
        out_shape=jax.ShapeDtypeStruct((N_TOK, D_SAE), jnp.float32),
    )(x, W, b2)


def _sc_body(pre_hbm, out_hbm, row_v, out_v, hist_v):
    cid = lax.axis_index("c")
    sid = lax.axis_index("s")
    wid = sid * 2 + cid

    lane = lax.iota(jnp.int32, 16)
    lane_base = lane * 256
    ones = jnp.ones((16,), jnp.int32)
    zeros_i = jnp.zeros((16,), jnp.int32)
    zeros_f = jnp.zeros((16,), jnp.float32)

    def load_m(i):
        v = row_v[pl.ds(i * 16, 16)]
        bits = lax.bitcast_convert_type(v, jnp.int32)
        m = bits ^ (lax.shift_right_arithmetic(bits, 31) & jnp.int32(0x7FFFFFFF))
        return v, m

    def per_row(r, carry0):
        row = wid * RPW + r
        pltpu.sync_copy(pre_hbm.at[row], row_v)

        p = jnp.int32(0)
        k = jnp.int32(K)
        lo = jnp.int32(1)
        hi = jnp.int32(0x7FFFFFFF)

        for shift in (24, 16, 8, 0):
            def clr(i, c):
                hist_v[pl.ds(i * 16, 16)] = zeros_i
                return c
            lax.fori_loop(0, 256, clr, 0)

            def hpass(i, mx):
                v, m = load_m(i)
                msk = (m >= lo) & (m < hi)
                d = lax.shift_right_arithmetic(m, jnp.int32(shift)) & jnp.int32(0xFF)
                plsc.addupdate_scatter(hist_v, [lane_base + d], ones, mask=msk)
                return jnp.maximum(mx, jnp.where(msk, m, jnp.int32(NEG_INF_I32)))
            mxv = lax.fori_loop(0, NV, hpass, jnp.full((16,), NEG_INF_I32, jnp.int32))

            if shift == 24:
                rowmax = jnp.max(mxv)
                c0 = jnp.where(rowmax >= 1,
                               lax.shift_right_arithmetic(rowmax, 28),
                               jnp.int32(-1))
            else:
                c0 = jnp.int32(15)

            def cond(st):
                c, carry, found, pp, kk = st
                return jnp.logical_and(jnp.logical_not(found), c >= 0)

            def body(st):
                c, carry, found, pp, kk = st
                t16 = zeros_i
                for l in range(16):
                    t16 = t16 + hist_v[pl.ds(l * 256 + c * 16, 16)]
                total = jnp.sum(t16)
                cs = jnp.cumsum(t16)
                s = total - cs + t16
                cumtop = carry + s
                found_now = (carry + total) >= kk
                mask2 = cumtop >= kk
                nm = jnp.sum(mask2.astype(jnp.int32))
                dstar = c * 16 + nm - 1
                cnt_excl = jnp.maximum(
                    jnp.max(jnp.where(mask2, jnp.int32(0), cumtop)), carry)
                pp2 = jnp.where(found_now, pp | (dstar << shift), pp)
                kk2 = jnp.where(found_now, kk - cnt_excl, kk)
                carry2 = jnp.where(found_now, carry, carry + total)
                return (c - 1, carry2, jnp.logical_or(found, found_now), pp2, kk2)

            cE, carryE, foundE, p, k = lax.while_loop(
                cond, body, (c0, jnp.int32(0), jnp.bool_(False), p, k))
            lo = jnp.where(foundE, jnp.maximum(p, 1), lo)
            hi = jnp.where(foundE, p + (jnp.int32(1) << shift), hi)

        t_m = jnp.maximum(p, 1)

        def wpass(i, c):
            v, m = load_m(i)
            out_v[pl.ds(i * 16, 16)] = jnp.where(m >= t_m, v, zeros_f)
            return c
        lax.fori_loop(0, NV, wpass, 0)

        pltpu.sync_copy(out_v, out_hbm.at[row])
        return carry0

    lax.fori_loop(0, RPW, per_row, 0)


def _sc_select(pre):
    mesh = plsc.VectorSubcoreMesh(core_axis_name="c", subcore_axis_name="s")
    f = functools.partial(
        pl.kernel,
        out_type=jax.ShapeDtypeStruct((N_TOK, D_SAE), jnp.float32),
        mesh=mesh,
        compiler_params=pltpu.CompilerParams(needs_layout_passes=False),
        scratch_types=[
            pltpu.VMEM((D_SAE,), jnp.float32),
            pltpu.VMEM((D_SAE,), jnp.float32),
            pltpu.VMEM((16 * 256,), jnp.int32),
        ],
    )(_sc_body)
    return f(pre)


def kernel(x, W, b):
    pre = _matmul(x, W, b)
    return _sc_select(pre)


# speedup vs baseline: 4.6571x; 1.9314x over previous
import functools

import jax
import jax.numpy as jnp
from jax import lax
from jax.experimental import pallas as pl
from jax.experimental.pallas import tpu as pltpu
from jax.experimental.pallas import tpu_sc as plsc

D_MODEL = 768
D_SAE = 24576
N_TOK = 4096
K = 64

BR = 1024
BC = 1024

NW = 32
RPW = N_TOK // NW
NV = D_SAE // 16
NEG_INF_I32 = -2147483648


def _mm_body(x_ref, w_ref, b_ref, o_ref):
    o_ref[...] = jax.lax.dot_general(
        x_ref[...], w_ref[...], (((1,), (1,)), ((), ())),
        preferred_element_type=jnp.float32) + b_ref[...]


def _matmul(x, W, b):
    nr = N_TOK // BR
    nc = D_SAE // BC
    b2 = b.reshape(1, D_SAE)
    return pl.pallas_call(
        _mm_body,
        grid=(nr, nc),
        in_specs=[
            pl.BlockSpec((BR, D_MODEL), lambda r, c: (r, 0)),
            pl.BlockSpec((BC, D_MODEL), lambda r, c: (c, 0)),
            pl.BlockSpec((1, BC), lambda r, c: (0, c)),
        ],
        out_specs=pl.BlockSpec((BR, BC), lambda r, c: (r, c)),
        out_shape=jax.ShapeDtypeStruct((N_TOK, D_SAE), jnp.float32),
    )(x, W, b2)


CAPL = 512
NSLOT = 4


def _sc_body(pre_hbm, out_hbm, row_v, out_v, hist_v, cand_v):
    cid = lax.axis_index("c")
    sid = lax.axis_index("s")
    wid = sid * 2 + cid

    lane = lax.iota(jnp.int32, 16)
    lane128 = lane * 128
    lane256 = lane * 256
    ones = jnp.ones((16,), jnp.int32)
    zeros_i = jnp.zeros((16,), jnp.int32)
    zeros_f = jnp.zeros((16,), jnp.float32)

    def to_m(v):
        bits = lax.bitcast_convert_type(v, jnp.int32)
        return bits ^ (lax.shift_right_arithmetic(bits, 31) & jnp.int32(0x7FFFFFFF))

    def scan_digits(c0, chunk_sum, kk_in, p_in, shift):
        def cond(st):
            c, carry, found, pp, kk = st
            return jnp.logical_and(jnp.logical_not(found), c >= 0)

        def body(st):
            c, carry, found, pp, kk = st
            t16 = chunk_sum(c)
            total = jnp.sum(t16)
            cs = jnp.cumsum(t16)
            s = total - cs + t16
            cumtop = carry + s
            found_now = (carry + total) >= kk
            mask2 = cumtop >= kk
            nm = jnp.sum(mask2.astype(jnp.int32))
            dstar = c * 16 + nm - 1
            cnt_excl = jnp.maximum(
                jnp.max(jnp.where(mask2, jnp.int32(0), cumtop)), carry)
            pp2 = jnp.where(found_now, pp | (dstar << shift), pp)
            kk2 = jnp.where(found_now, kk - cnt_excl, kk)
            carry2 = jnp.where(found_now, carry, carry + total)
            return (c - 1, carry2, jnp.logical_or(found, found_now), pp2, kk2)

        cE, carryE, foundE, p2, k2 = lax.while_loop(
            cond, body, (c0, jnp.int32(0), jnp.bool_(False), p_in, kk_in))
        return foundE, p2, k2

    def per_row(r, carry0):
        row = wid * RPW + r
        pltpu.sync_copy(pre_hbm.at[row], row_v)

        def clr1(i, c):
            hist_v[pl.ds(i * 16, 16)] = zeros_i
            return c
        lax.fori_loop(0, NSLOT * 128, clr1, 0, unroll=8)

        def hpass(i, mx):
            v = row_v[pl.ds(i * 16, 16)]
            m = to_m(v)
            msk = m >= 1
            d = lax.shift_right_arithmetic(m, 24) & jnp.int32(0x7F)
            slot = (i & 3) * 2048
            plsc.addupdate_scatter(hist_v, [slot + lane128 + d], ones, mask=msk)
            return jnp.maximum(mx, jnp.where(msk, m, jnp.int32(NEG_INF_I32)))
        mxv = lax.fori_loop(0, NV, hpass,
                            jnp.full((16,), NEG_INF_I32, jnp.int32), unroll=4)
        rowmax = jnp.max(mxv)
        c0 = jnp.where(rowmax >= 1,
                       lax.shift_right_arithmetic(rowmax, 28), jnp.int32(-1))

        def chunk_sum1(c):
            t16 = zeros_i
            for slot in range(NSLOT):
                for l in range(16):
                    t16 = t16 + hist_v[pl.ds(slot * 2048 + l * 128 + c * 16, 16)]
            return t16

        found1, p, k = scan_digits(c0, chunk_sum1, jnp.int32(K), jnp.int32(0), 24)
        lo = jnp.where(found1, jnp.maximum(p, 1), jnp.int32(1))
        hi = jnp.where(found1, p + jnp.int32(1 << 24), lo)

        def wpass(i, ptr):
            v = row_v[pl.ds(i * 16, 16)]
            m = to_m(v)
            keep = m >= lo
            out_v[pl.ds(i * 16, 16)] = jnp.where(keep, v, zeros_f)
            cmask = keep & (m < hi) & (ptr < CAPL)
            plsc.store_scatter(cand_v, [ptr * 16 + lane], i * 16 + lane,
                               mask=cmask)
            return ptr + cmask.astype(jnp.int32)
        cnt_vec = lax.fori_loop(0, NV, wpass, zeros_i, unroll=4)
        maxcnt = jnp.max(cnt_vec)

        def chunk_sum2(c):
            t16 = zeros_i
            for l in range(16):
                t16 = t16 + hist_v[pl.ds(l * 256 + c * 16, 16)]
            return t16

        for shift in (16, 8, 0):
            def clr2(i, c):
                hist_v[pl.ds(i * 16, 16)] = zeros_i
                return c
            lax.fori_loop(0, 256, clr2, 0, unroll=8)

            def cpass(j, c, _shift=shift, _lo=lo, _hi=hi):
                valid = cnt_vec > j
                idxv = cand_v[pl.ds(j * 16, 16)]
                vv = plsc.load_gather(row_v, [idxv], mask=valid)
                m = to_m(vv)
                msk = valid & (m >= _lo) & (m < _hi)
                d = lax.shift_right_arithmetic(m, _shift) & jnp.int32(0xFF)
                plsc.addupdate_scatter(hist_v, [lane256 + d], ones, mask=msk)
                return c
            lax.fori_loop(0, maxcnt, cpass, 0)

            foundR, p, k = scan_digits(jnp.int32(15), chunk_sum2, k, p, shift)
            lo = jnp.where(foundR, jnp.maximum(p, 1), lo)
            hi = jnp.where(foundR, p + jnp.int32(1 << shift), hi)

        t_m = jnp.maximum(p, 1)

        def fpass(j, c):
            valid = cnt_vec > j
            idxv = cand_v[pl.ds(j * 16, 16)]
            vv = plsc.load_gather(row_v, [idxv], mask=valid)
            m = to_m(vv)
            loser = valid & (m < t_m)
            plsc.store_scatter(out_v, [idxv], zeros_f, mask=loser)
            return c
        lax.fori_loop(0, maxcnt, fpass, 0)

        pltpu.sync_copy(out_v, out_hbm.at[row])
        return carry0

    lax.fori_loop(0, RPW, per_row, 0)


def _sc_select(pre):
    mesh = plsc.VectorSubcoreMesh(core_axis_name="c", subcore_axis_name="s")
    f = functools.partial(
        pl.kernel,
        out_type=jax.ShapeDtypeStruct((N_TOK, D_SAE), jnp.float32),
        mesh=mesh,
        compiler_params=pltpu.CompilerParams(needs_layout_passes=False),
        scratch_types=[
            pltpu.VMEM((D_SAE,), jnp.float32),
            pltpu.VMEM((D_SAE,), jnp.float32),
            pltpu.VMEM((NSLOT * 16 * 128,), jnp.int32),
            pltpu.VMEM((CAPL * 16,), jnp.int32),
        ],
    )(_sc_body)
    return f(pre)


def kernel(x, W, b):
    pre = _matmul(x, W, b)
    return _sc_select(pre)


# speedup vs baseline: 9.6680x; 2.0760x over previous
import functools

import jax
import jax.numpy as jnp
from jax import lax
from jax.experimental import pallas as pl
from jax.experimental.pallas import tpu as pltpu
from jax.experimental.pallas import tpu_sc as plsc

D_MODEL = 768
D_SAE = 24576
N_TOK = 4096
K = 64

BR = 1024
BC = 1024

NW = 32
RPW = N_TOK // NW
NV = D_SAE // 16
NEG_INF_I32 = -2147483648


def _mm_body(x_ref, w_ref, b_ref, o_ref):
    o_ref[...] = jax.lax.dot_general(
        x_ref[...], w_ref[...], (((1,), (1,)), ((), ())),
        preferred_element_type=jnp.float32) + b_ref[...]


def _matmul(x, W, b):
    nr = N_TOK // BR
    nc = D_SAE // BC
    b2 = b.reshape(1, D_SAE)
    return pl.pallas_call(
        _mm_body,
        grid=(nr, nc),
        in_specs=[
            pl.BlockSpec((BR, D_MODEL), lambda r, c: (r, 0)),
            pl.BlockSpec((BC, D_MODEL), lambda r, c: (c, 0)),
            pl.BlockSpec((1, BC), lambda r, c: (0, c)),
        ],
        out_specs=pl.BlockSpec((BR, BC), lambda r, c: (r, c)),
        out_shape=jax.ShapeDtypeStruct((N_TOK, D_SAE), jnp.float32),
    )(x, W, b2)


CAPL = 512
NSLOT = 4


def _sc_body(pre_hbm, out_hbm, row_v, out_v, hist_v, cand_v):
    cid = lax.axis_index("c")
    sid = lax.axis_index("s")
    wid = sid * 2 + cid

    lane = lax.iota(jnp.int32, 16)
    lane128 = lane * 128
    lane256 = lane * 256
    ones = jnp.ones((16,), jnp.int32)
    zeros_i = jnp.zeros((16,), jnp.int32)
    zeros_f = jnp.zeros((16,), jnp.float32)

    def to_m(v):
        bits = lax.bitcast_convert_type(v, jnp.int32)
        return bits ^ (lax.shift_right_arithmetic(bits, 31) & jnp.int32(0x7FFFFFFF))

    def scan_digits(c0, chunk_sum, kk_in, p_in, shift):
        def cond(st):
            c, carry, found, pp, kk = st
            return jnp.logical_and(jnp.logical_not(found), c >= 0)

        def body(st):
            c, carry, found, pp, kk = st
            t16 = chunk_sum(c)
            total = jnp.sum(t16)
            cs = jnp.cumsum(t16)
            s = total - cs + t16
            cumtop = carry + s
            found_now = (carry + total) >= kk
            mask2 = cumtop >= kk
            nm = jnp.sum(mask2.astype(jnp.int32))
            dstar = c * 16 + nm - 1
            cnt_excl = jnp.maximum(
                jnp.max(jnp.where(mask2, jnp.int32(0), cumtop)), carry)
            pp2 = jnp.where(found_now, pp | (dstar << shift), pp)
            kk2 = jnp.where(found_now, kk - cnt_excl, kk)
            carry2 = jnp.where(found_now, carry, carry + total)
            return (c - 1, carry2, jnp.logical_or(found, found_now), pp2, kk2)

        cE, carryE, foundE, p2, k2 = lax.while_loop(
            cond, body, (c0, jnp.int32(0), jnp.bool_(False), p_in, kk_in))
        return foundE, p2, k2

    def per_row(r, carry0):
        row = wid * RPW + r
        pltpu.sync_copy(pre_hbm.at[row], row_v)

        @plsc.parallel_loop(0, NSLOT * 128, unroll=8)
        def _clr1(i):
            hist_v[pl.ds(i * 16, 16)] = zeros_i

        @plsc.parallel_loop(0, NV, unroll=8,
                            carry=jnp.full((16,), NEG_INF_I32, jnp.int32))
        def mxv(i, mx):
            v = row_v[pl.ds(i * 16, 16)]
            m = to_m(v)
            msk = m >= 1
            d = lax.shift_right_arithmetic(m, 24) & jnp.int32(0x7F)
            slot = (i & 3) * 2048
            plsc.addupdate_scatter(hist_v, [slot + lane128 + d], ones, mask=msk)
            return jnp.maximum(mx, jnp.where(msk, m, jnp.int32(NEG_INF_I32)))
        rowmax = jnp.max(mxv)
        c0 = jnp.where(rowmax >= 1,
                       lax.shift_right_arithmetic(rowmax, 28), jnp.int32(-1))

        def chunk_sum1(c):
            t16 = zeros_i
            for slot in range(NSLOT):
                for l in range(16):
                    t16 = t16 + hist_v[pl.ds(slot * 2048 + l * 128 + c * 16, 16)]
            return t16

        found1, p, k = scan_digits(c0, chunk_sum1, jnp.int32(K), jnp.int32(0), 24)
        lo = jnp.where(found1, jnp.maximum(p, 1), jnp.int32(1))
        hi = jnp.where(found1, p + jnp.int32(1 << 24), lo)

        @plsc.parallel_loop(0, NV, unroll=8, carry=zeros_i)
        def cnt_vec(i, ptr):
            v = row_v[pl.ds(i * 16, 16)]
            m = to_m(v)
            keep = m >= lo
            out_v[pl.ds(i * 16, 16)] = jnp.where(keep, v, zeros_f)
            cmask = keep & (m < hi) & (ptr < CAPL)
            plsc.store_scatter(cand_v, [ptr * 16 + lane], i * 16 + lane,
                               mask=cmask)
            return ptr + cmask.astype(jnp.int32)
        maxcnt = jnp.max(cnt_vec)

        def chunk_sum2(c):
            t16 = zeros_i
            for l in range(16):
                t16 = t16 + hist_v[pl.ds(l * 256 + c * 16, 16)]
            return t16

        for shift in (16, 8, 0):
            @plsc.parallel_loop(0, 256, unroll=8)
            def _clr2(i):
                hist_v[pl.ds(i * 16, 16)] = zeros_i

            def cpass(j, c, _shift=shift, _lo=lo, _hi=hi):
                valid = cnt_vec > j
                idxv = cand_v[pl.ds(j * 16, 16)]
                vv = plsc.load_gather(row_v, [idxv], mask=valid)
                m = to_m(vv)
                msk = valid & (m >= _lo) & (m < _hi)
                d = lax.shift_right_arithmetic(m, _shift) & jnp.int32(0xFF)
                plsc.addupdate_scatter(hist_v, [lane256 + d], ones, mask=msk)
                return c
            lax.fori_loop(0, maxcnt, cpass, 0)

            foundR, p, k = scan_digits(jnp.int32(15), chunk_sum2, k, p, shift)
            lo = jnp.where(foundR, jnp.maximum(p, 1), lo)
            hi = jnp.where(foundR, p + jnp.int32(1 << shift), hi)

        t_m = jnp.maximum(p, 1)

        def fpass(j, c):
            valid = cnt_vec > j
            idxv = cand_v[pl.ds(j * 16, 16)]
            vv = plsc.load_gather(row_v, [idxv], mask=valid)
            m = to_m(vv)
            loser = valid & (m < t_m)
            plsc.store_scatter(out_v, [idxv], zeros_f, mask=loser)
            return c
        lax.fori_loop(0, maxcnt, fpass, 0)

        pltpu.sync_copy(out_v, out_hbm.at[row])
        return carry0

    lax.fori_loop(0, RPW, per_row, 0)


def _sc_select(pre):
    mesh = plsc.VectorSubcoreMesh(core_axis_name="c", subcore_axis_name="s")
    f = functools.partial(
        pl.kernel,
        out_type=jax.ShapeDtypeStruct((N_TOK, D_SAE), jnp.float32),
        mesh=mesh,
        compiler_params=pltpu.CompilerParams(needs_layout_passes=False),
        scratch_types=[
            pltpu.VMEM((D_SAE,), jnp.float32),
            pltpu.VMEM((D_SAE,), jnp.float32),
            pltpu.VMEM((NSLOT * 16 * 128,), jnp.int32),
            pltpu.VMEM((CAPL * 16,), jnp.int32),
        ],
    )(_sc_body)
    return f(pre)


def kernel(x, W, b):
    pre = _matmul(x, W, b)
    return _sc_select(pre)
